# Initial kernel scaffold; baseline (speedup 1.0000x reference)
import jax
import jax.numpy as jnp
from jax.experimental import pallas as pl


def kernel(hidden_states, top_k_weights, entropy, W_sig, b_sig, ln_g, ln_b, W_hid, W_rh1, b_rh1, W_rh2, b_rh2):
    raise NotImplementedError("write your pallas kernel here")


# speedup vs baseline: 2.0050x; 2.0050x over previous
import functools

import jax
import jax.numpy as jnp
from jax.experimental import pallas as pl
from jax.experimental.pallas import tpu as pltpu


def _router_body(x_ref, sig_ref, wsig_ref, bsig_ref, g_ref, b_ref,
                 whid_ref, w1_ref, b1_ref, w2_ref, b2_ref, out_ref):
    f32 = jnp.float32
    eps = 1e-5
    x = x_ref[...]
    D = x.shape[1]
    mu = jnp.mean(x, axis=1, keepdims=True)
    d = x - mu
    var = jnp.mean(d * d, axis=1, keepdims=True)
    xn = d * jax.lax.rsqrt(var + eps) * g_ref[...] + b_ref[...]
    he = xn @ whid_ref[...]
    he = he * jax.nn.sigmoid(he)

    sig = sig_ref[...]
    se = sig @ wsig_ref[...] + bsig_ref[...]
    se = se * jax.nn.sigmoid(se)

    comb = jnp.concatenate([se, he], axis=1)
    h1 = comb @ w1_ref[...] + b1_ref[...]
    h1 = h1 * jax.nn.sigmoid(h1)
    logits = h1 @ w2_ref[...] + b2_ref[...]

    TB, E = logits.shape
    iota = jax.lax.broadcasted_iota(jnp.int32, (TB, E), 1)
    m1 = jnp.max(logits, axis=1, keepdims=True)
    i1 = jnp.min(jnp.where(logits == m1, iota, E), axis=1, keepdims=True)
    is1 = iota == i1
    l2 = jnp.where(is1, -jnp.inf, logits)
    m2 = jnp.max(l2, axis=1, keepdims=True)
    i2 = jnp.min(jnp.where(l2 == m2, iota, E), axis=1, keepdims=True)
    is2 = iota == i2
    p1 = 1.0 / (1.0 + jnp.exp(m2 - m1))
    p2 = 1.0 - p1
    zero = jnp.zeros((), f32)
    out_ref[...] = jnp.where(is1, p1, zero) + jnp.where(is2, p2, zero)


@jax.jit
def kernel(hidden_states, top_k_weights, entropy, W_sig, b_sig, ln_g, ln_b,
           W_hid, W_rh1, b_rh1, W_rh2, b_rh2):
    B, S, D = hidden_states.shape
    K = top_k_weights.shape[-1]
    E = W_rh2.shape[1]
    N = B * S
    x = hidden_states.reshape(N, D)
    sig = jnp.concatenate(
        [top_k_weights.reshape(N, K), entropy.reshape(N, 1)], axis=1)

    TB = 512
    grid = (N // TB,)

    def tok(i):
        return (i, 0)

    def rep(i):
        return (0, 0)

    out = pl.pallas_call(
        _router_body,
        grid=grid,
        in_specs=[
            pl.BlockSpec((TB, D), tok),
            pl.BlockSpec((TB, K + 1), tok),
            pl.BlockSpec(W_sig.shape, rep),
            pl.BlockSpec((1, b_sig.shape[0]), rep),
            pl.BlockSpec((1, D), rep),
            pl.BlockSpec((1, D), rep),
            pl.BlockSpec(W_hid.shape, rep),
            pl.BlockSpec(W_rh1.shape, rep),
            pl.BlockSpec((1, b_rh1.shape[0]), rep),
            pl.BlockSpec(W_rh2.shape, rep),
            pl.BlockSpec((1, b_rh2.shape[0]), rep),
        ],
        out_specs=pl.BlockSpec((TB, E), tok),
        out_shape=jax.ShapeDtypeStruct((N, E), jnp.float32),
        compiler_params=pltpu.CompilerParams(
            dimension_semantics=("arbitrary",),
        ),
    )(x, sig, W_sig, b_sig.reshape(1, -1), ln_g.reshape(1, -1),
      ln_b.reshape(1, -1), W_hid, W_rh1, b_rh1.reshape(1, -1), W_rh2,
      b_rh2.reshape(1, -1))
    return out.reshape(B, S, E)
